# baseline (speedup 1.0000x reference)
import functools

import jax
import jax.numpy as jnp
from jax import lax
from jax.experimental import pallas as pl
from jax.experimental.pallas import tpu as pltpu
from jax.experimental.pallas import tpu_sc as plsc

B = 16384
D = 32
LANES = 16
NUM_WORKERS = 32
B_PER_W = B // NUM_WORKERS
IDX_CHUNK = 128
N_CHUNKS = B_PER_W // IDX_CHUNK
N_GROUPS = B_PER_W // LANES


def _sc_body(t_idx_hbm, c_idx_hbm, t_tab_hbm, c_tab_hbm, out_hbm,
             t_idx_v, c_idx_v, t_rows_v, c_rows_v, out_v, sem):
    nc = 2
    wid = lax.axis_index("s") * nc + lax.axis_index("c")
    blk = wid * N_CHUNKS

    pltpu.sync_copy(t_idx_hbm.at[pl.ds(blk, N_CHUNKS)], t_idx_v)
    pltpu.sync_copy(c_idx_hbm.at[pl.ds(blk, N_CHUNKS)], c_idx_v)

    cps = []
    for j in range(N_CHUNKS):
        cps.append(pltpu.async_copy(
            t_tab_hbm.at[t_idx_v.at[j]],
            t_rows_v.at[pl.ds(j * IDX_CHUNK, IDX_CHUNK)], sem))
        cps.append(pltpu.async_copy(
            c_tab_hbm.at[c_idx_v.at[j]],
            c_rows_v.at[pl.ds(j * IDX_CHUNK, IDX_CHUNK)], sem))
    for cp in cps:
        cp.wait()

    lanes = lax.iota(jnp.int32, LANES)
    cols = [jnp.full((LANES,), d, jnp.int32) for d in range(D)]

    def group(g, _):
        rows = jnp.int32(LANES) * g + lanes
        accs = [jnp.zeros((LANES,), jnp.float32) for _ in range(4)]
        for d in range(D):
            tv = plsc.load_gather(t_rows_v, [rows, cols[d]])
            cv = plsc.load_gather(c_rows_v, [rows, cols[d]])
            accs[d % 4] = accs[d % 4] + tv * cv
        out_v[pl.ds(g * LANES, LANES)] = (accs[0] + accs[1]) + (accs[2] + accs[3])
        return 0

    lax.fori_loop(0, N_GROUPS, group, 0)

    pltpu.sync_copy(out_v, out_hbm.at[pl.ds(wid * B_PER_W, B_PER_W)])


@jax.jit
def _run(t_idx, c_idx, t_tab, c_tab):
    mesh = plsc.VectorSubcoreMesh(core_axis_name="c", subcore_axis_name="s")
    return pl.kernel(
        _sc_body,
        out_type=jax.ShapeDtypeStruct((B,), jnp.float32),
        mesh=mesh,
        compiler_params=pltpu.CompilerParams(
            needs_layout_passes=False, use_tc_tiling_on_sc=False),
        scratch_types=[
            pltpu.VMEM((N_CHUNKS, IDX_CHUNK), jnp.int32),
            pltpu.VMEM((N_CHUNKS, IDX_CHUNK), jnp.int32),
            pltpu.VMEM((B_PER_W, D), jnp.float32),
            pltpu.VMEM((B_PER_W, D), jnp.float32),
            pltpu.VMEM((B_PER_W,), jnp.float32),
            pltpu.SemaphoreType.DMA,
        ],
    )(t_idx, c_idx, t_tab, c_tab)


def kernel(t_kmer, c_kmer, label, T_weight, C_weight):
    del label
    t_idx = t_kmer.astype(jnp.int32).reshape(B // IDX_CHUNK, IDX_CHUNK)
    c_idx = c_kmer.astype(jnp.int32).reshape(B // IDX_CHUNK, IDX_CHUNK)
    return _run(t_idx, c_idx, T_weight, C_weight)
